# baseline (speedup 1.0000x reference)
import functools

import jax
import jax.numpy as jnp
from jax import lax
from jax.experimental import pallas as pl
from jax.experimental.pallas import tpu as pltpu
from jax.experimental.pallas import tpu_sc as plsc

N_NODES = 10000
D = 128
HW = D // 2
NC = 2
NS = 16
NW = NC * NS
CHUNK = 64
CPT = 160
PH = 16
NPH = CPT // PH
E_PAD = NW * CPT * CHUNK
N_PAD = 10240
ROWS_PER_TILE = N_PAD // NS

_DNUMS = lax.GatherDimensionNumbers(
    offset_dims=(), collapsed_slice_dims=(0,), start_index_map=(0,))


def _mm_body(x_ref, w_ref, o_ref):
    o_ref[...] = jnp.dot(x_ref[...], w_ref[...],
                         preferred_element_type=jnp.float32)


def _combine_body(p_ref, o_ref):
    o_ref[...] = jnp.maximum(p_ref[0] + p_ref[1], 0.0)


_sc_mesh = plsc.VectorSubcoreMesh(core_axis_name="c", subcore_axis_name="s")


@functools.partial(
    pl.kernel,
    mesh=_sc_mesh,
    out_type=jax.ShapeDtypeStruct((NC, N_PAD, D), jnp.float32),
    scratch_types=[
        pltpu.VMEM((PH, CHUNK), jnp.int32),
        pltpu.VMEM((PH, CHUNK), jnp.int32),
        pltpu.VMEM((PH, CHUNK), jnp.int32),
        pltpu.VMEM((PH, CHUNK), jnp.int32),
        pltpu.VMEM((PH, CHUNK), jnp.float32),
        pltpu.VMEM((PH, CHUNK), jnp.float32),
        pltpu.VMEM((CHUNK, D), jnp.float32),
        pltpu.VMEM((CHUNK, D), jnp.float32),
        pltpu.VMEM((CHUNK, D), jnp.float32),
        pltpu.VMEM((CHUNK, D), jnp.float32),
        pltpu.VMEM_SHARED((N_PAD, D), jnp.float32),
        pltpu.SemaphoreType.DMA,
        pltpu.SemaphoreType.DMA,
        pltpu.SemaphoreType.DMA,
        pltpu.SemaphoreType.DMA,
        pltpu.SemaphoreType.DMA,
    ],
)
def _sc_scatter(hidden_hbm, rows_hbm, cols_hbm, w_hbm, zeros_hbm, out_hbm,
                rows_s0, rows_s1, cols_s0, cols_s1, w_s0, w_s1,
                pbuf0, pbuf1, sbuf0, sbuf1, acc,
                gsem0, gsem1, ssem0, ssem1, tsem):
    c = lax.axis_index("c")
    s = lax.axis_index("s")
    wid = s * NC + c

    rows_s = (rows_s0, rows_s1)
    cols_s = (cols_s0, cols_s1)
    w_s = (w_s0, w_s1)
    pbufs = (pbuf0, pbuf1)
    sbufs = (sbuf0, sbuf1)
    gsems = (gsem0, gsem1)
    ssems = (ssem0, ssem1)

    pltpu.sync_copy(zeros_hbm.at[pl.ds(s * ROWS_PER_TILE, ROWS_PER_TILE)],
                    acc.at[pl.ds(s * ROWS_PER_TILE, ROWS_PER_TILE)])
    plsc.subcore_barrier()

    def stage_srcs(h):
        return (rows_hbm.at[wid, pl.ds(h * PH, PH)],
                cols_hbm.at[wid, pl.ds(h * PH, PH)],
                w_hbm.at[wid, pl.ds(h * PH, PH)])

    def stage(h, sl, sem):
        for a, d in zip(stage_srcs(h), (rows_s[sl], cols_s[sl], w_s[sl])):
            if sem is None:
                pltpu.sync_copy(a, d)
            else:
                pltpu.async_copy(a, d, sem)

    def scale_chunk(gg, b, w_sl):
        pbuf, sbuf = pbufs[b], sbufs[b]

        def edge_body(e, carry):
            base = (e // 16) * 16
            w16 = w_sl[gg, pl.ds(base, 16)]
            wsplat = lax.gather(
                w16, jnp.full((16, 1), e - base, jnp.int32), _DNUMS,
                slice_sizes=(1,),
                mode=lax.GatherScatterMode.PROMISE_IN_BOUNDS)
            for k in range(8):
                sbuf[e, pl.ds(16 * k, 16)] = wsplat * pbuf[e, pl.ds(16 * k, 16)]
            return carry

        lax.fori_loop(0, CHUNK, edge_body, 0)

    stage(0, 0, None)
    stage(1, 1, tsem)

    for b in range(2):
        pltpu.async_copy(hidden_hbm.at[cols_s0.at[b]], pbufs[b], gsems[b])

    for h in range(NPH):
        sl = h % 2
        nsl = 1 - sl
        rows_sl, cols_sl, w_sl = rows_s[sl], cols_s[sl], w_s[sl]

        if h > 0:
            for b in range(2):
                pltpu.make_async_copy(
                    sbufs[b], acc.at[rows_s[nsl].at[PH - 2 + b]],
                    ssems[b]).wait()
            if h + 1 < NPH:
                stage(h + 1, nsl, tsem)

        def pair_body(i, carry):
            for b in range(2):
                g = 2 * i + b
                pbuf, sbuf = pbufs[b], sbufs[b]
                pltpu.make_async_copy(
                    hidden_hbm.at[cols_sl.at[g]], pbuf, gsems[b]).wait()
                @pl.when(i >= 1)
                def _():
                    pltpu.make_async_copy(
                        sbuf, acc.at[rows_sl.at[g]], ssems[b]).wait()
                scale_chunk(g, b, w_sl)
                pltpu.async_copy(
                    sbuf, acc.at[rows_sl.at[g]], ssems[b], add=True)
                pltpu.async_copy(
                    hidden_hbm.at[cols_sl.at[g + 2]], pbuf, gsems[b])
            return carry

        lax.fori_loop(0, PH // 2 - 1, pair_body, 0)

        for b in range(2):
            g = PH - 2 + b
            pbuf, sbuf = pbufs[b], sbufs[b]
            pltpu.make_async_copy(
                hidden_hbm.at[cols_sl.at[g]], pbuf, gsems[b]).wait()
            pltpu.make_async_copy(
                sbuf, acc.at[rows_sl.at[g - 2]], ssems[b]).wait()
            scale_chunk(g, b, w_sl)
            pltpu.async_copy(sbuf, acc.at[rows_sl.at[g]], ssems[b], add=True)
            if h + 1 < NPH:
                if b == 0:
                    for a, d in zip(
                            stage_srcs(h + 1),
                            (rows_s[nsl], cols_s[nsl], w_s[nsl])):
                        pltpu.make_async_copy(a, d, tsem).wait()
                pltpu.async_copy(
                    hidden_hbm.at[cols_s[nsl].at[b]], pbufs[b], gsems[b])

    for b in range(2):
        pltpu.make_async_copy(
            sbufs[b], acc.at[rows_s[(NPH - 1) % 2].at[PH - 2 + b]],
            ssems[b]).wait()
    plsc.subcore_barrier()

    pltpu.sync_copy(acc.at[pl.ds(s * ROWS_PER_TILE, ROWS_PER_TILE)],
                    out_hbm.at[c, pl.ds(s * ROWS_PER_TILE, ROWS_PER_TILE)])


def kernel(X, edge_index, edge_weight, W):
    X_flat = X.reshape(N_NODES, D)

    hidden_pk = pl.pallas_call(
        _mm_body,
        grid=(10,),
        in_specs=[
            pl.BlockSpec((1000, D), lambda i: (i, 0)),
            pl.BlockSpec((D, D), lambda i: (0, 0)),
        ],
        out_specs=pl.BlockSpec((1000, D), lambda i: (i, 0)),
        out_shape=jax.ShapeDtypeStruct((N_NODES, D), jnp.float32),
    )(X_flat, W)

    e = edge_weight.shape[0]
    pad = E_PAD - e
    rows = jnp.concatenate(
        [edge_index[0].astype(jnp.int32), jnp.zeros((pad,), jnp.int32)]
    ).reshape(NW, CPT, CHUNK)
    cols = jnp.concatenate(
        [edge_index[1].astype(jnp.int32), jnp.zeros((pad,), jnp.int32)]
    ).reshape(NW, CPT, CHUNK)
    w_e = jnp.concatenate(
        [edge_weight.astype(jnp.float32), jnp.zeros((pad,), jnp.float32)]
    ).reshape(NW, CPT, CHUNK)
    zeros = jnp.zeros((N_PAD, D), jnp.float32)

    partials = _sc_scatter(hidden_pk, rows, cols, w_e, zeros)

    out = pl.pallas_call(
        _combine_body,
        grid=(10,),
        in_specs=[pl.BlockSpec((NC, 1024, D), lambda i: (0, i, 0))],
        out_specs=pl.BlockSpec((1024, D), lambda i: (i, 0)),
        out_shape=jax.ShapeDtypeStruct((N_PAD, D), jnp.float32),
    )(partials)

    return out[:N_NODES].reshape(1, N_NODES, D)


# speedup vs baseline: 1.1476x; 1.1476x over previous
import functools

import jax
import jax.numpy as jnp
from jax import lax
from jax.experimental import pallas as pl
from jax.experimental.pallas import tpu as pltpu
from jax.experimental.pallas import tpu_sc as plsc

N_NODES = 10000
D = 128
NC = 2
NS = 16
NW = NC * NS
CHUNK = 128
CPT = 80
HALF = CPT // 2
E_PAD = NW * CPT * CHUNK
N_PAD = 10240
ROWS_PER_TILE = N_PAD // NS


def _mm_body(x_ref, w_ref, o_ref):
    o_ref[...] = jnp.dot(x_ref[...], w_ref[...], preferred_element_type=jnp.float32)


def _combine_body(p_ref, o_ref):
    o_ref[...] = jnp.maximum(p_ref[0] + p_ref[1], 0.0)


_sc_mesh = plsc.VectorSubcoreMesh(core_axis_name="c", subcore_axis_name="s")


@functools.partial(
    pl.kernel,
    mesh=_sc_mesh,
    out_type=jax.ShapeDtypeStruct((NC, N_PAD, D), jnp.float32),
    scratch_types=[
        pltpu.VMEM((HALF, CHUNK), jnp.int32),
        pltpu.VMEM((HALF, CHUNK), jnp.int32),
        pltpu.VMEM((HALF, CHUNK), jnp.float32),
        pltpu.VMEM((CHUNK, D), jnp.float32),
        pltpu.VMEM((CHUNK, D), jnp.float32),
        pltpu.VMEM_SHARED((N_PAD, D), jnp.float32),
        pltpu.SemaphoreType.DMA,
        pltpu.SemaphoreType.DMA,
        pltpu.SemaphoreType.DMA,
        pltpu.SemaphoreType.DMA,
    ],
)
def _sc_scatter(hidden_hbm, rows_hbm, cols_hbm, w_hbm, out_hbm,
                rows_v, cols_v, w_v, buf0, buf1, acc, gsem0, gsem1, ssem0, ssem1):
    c = lax.axis_index("c")
    s = lax.axis_index("s")
    wid = s * NC + c

    def zrow(i, carry):
        for j in range(D // 16):
            buf0[i, pl.ds(j * 16, 16)] = jnp.zeros((16,), jnp.float32)
        return carry

    lax.fori_loop(0, CHUNK, zrow, 0)
    for j in range(ROWS_PER_TILE // CHUNK):
        pltpu.sync_copy(
            buf0, acc.at[pl.ds(s * ROWS_PER_TILE + j * CHUNK, CHUNK)])
    plsc.subcore_barrier()

    bufs = (buf0, buf1)
    gsems = (gsem0, gsem1)
    ssems = (ssem0, ssem1)
    npair = HALF // 2

    for h in range(2):
        pltpu.sync_copy(rows_hbm.at[wid, pl.ds(h * HALF, HALF)], rows_v)
        pltpu.sync_copy(cols_hbm.at[wid, pl.ds(h * HALF, HALF)], cols_v)
        pltpu.sync_copy(w_hbm.at[wid, pl.ds(h * HALF, HALF)], w_v)

        for b in range(2):
            pltpu.async_copy(hidden_hbm.at[cols_v.at[b]], bufs[b], gsems[b])

        def pair_body(i, carry):
            for b in range(2):
                g = 2 * i + b
                buf = bufs[b]
                pltpu.make_async_copy(
                    hidden_hbm.at[cols_v.at[g]], buf, gsems[b]).wait()

                def group_body(k, carry2):
                    w16 = w_v[g, pl.ds(k * 16, 16)]
                    for el in range(16):
                        wsplat = lax.gather(
                            w16,
                            jnp.full((16, 1), el, jnp.int32),
                            lax.GatherDimensionNumbers(
                                offset_dims=(), collapsed_slice_dims=(0,),
                                start_index_map=(0,)),
                            slice_sizes=(1,),
                            mode=lax.GatherScatterMode.PROMISE_IN_BOUNDS,
                        )
                        e = k * 16 + el
                        for j in range(D // 16):
                            buf[e, pl.ds(j * 16, 16)] = (
                                wsplat * buf[e, pl.ds(j * 16, 16)])
                    return carry2

                lax.fori_loop(0, CHUNK // 16, group_body, 0)

                pltpu.async_copy(buf, acc.at[rows_v.at[g]], ssems[b], add=True)

                @pl.when(i < npair - 1)
                def _():
                    pltpu.make_async_copy(
                        buf, acc.at[rows_v.at[g]], ssems[b]).wait()
                    pltpu.async_copy(
                        hidden_hbm.at[cols_v.at[g + 2]], buf, gsems[b])
            return carry

        lax.fori_loop(0, npair, pair_body, 0)

        for b in range(2):
            g = HALF - 2 + b
            pltpu.make_async_copy(bufs[b], acc.at[rows_v.at[g]], ssems[b]).wait()

    plsc.subcore_barrier()

    plsc.subcore_barrier()

    pltpu.sync_copy(acc.at[pl.ds(s * ROWS_PER_TILE, ROWS_PER_TILE)],
                    out_hbm.at[c, pl.ds(s * ROWS_PER_TILE, ROWS_PER_TILE)])


def kernel(X, edge_index, edge_weight, W):
    X_flat = X.reshape(N_NODES, D)

    hidden = pl.pallas_call(
        _mm_body,
        grid=(10,),
        in_specs=[
            pl.BlockSpec((1000, D), lambda i: (i, 0)),
            pl.BlockSpec((D, D), lambda i: (0, 0)),
        ],
        out_specs=pl.BlockSpec((1000, D), lambda i: (i, 0)),
        out_shape=jax.ShapeDtypeStruct((N_NODES, D), jnp.float32),
    )(X_flat, W)

    e = edge_weight.shape[0]
    pad = E_PAD - e
    rows = jnp.concatenate(
        [edge_index[0].astype(jnp.int32), jnp.zeros((pad,), jnp.int32)]
    ).reshape(NW, CPT, CHUNK)
    cols = jnp.concatenate(
        [edge_index[1].astype(jnp.int32), jnp.zeros((pad,), jnp.int32)]
    ).reshape(NW, CPT, CHUNK)
    w_e = jnp.concatenate(
        [edge_weight.astype(jnp.float32), jnp.zeros((pad,), jnp.float32)]
    ).reshape(NW, CPT, CHUNK)
    partials = _sc_scatter(hidden, rows, cols, w_e)

    out = pl.pallas_call(
        _combine_body,
        grid=(10,),
        in_specs=[pl.BlockSpec((NC, 1024, D), lambda i: (0, i, 0))],
        out_specs=pl.BlockSpec((1024, D), lambda i: (i, 0)),
        out_shape=jax.ShapeDtypeStruct((N_PAD, D), jnp.float32),
    )(partials)

    return out[:N_NODES].reshape(1, N_NODES, D)


# speedup vs baseline: 1.2866x; 1.1211x over previous
import functools

import jax
import jax.numpy as jnp
from jax import lax
from jax.experimental import pallas as pl
from jax.experimental.pallas import tpu as pltpu
from jax.experimental.pallas import tpu_sc as plsc

N_NODES = 10000
D = 128
NC = 2
NS = 16
NW = NC * NS
CHUNK = 128
CPT = 80
HALF = CPT // 2
E_PAD = NW * CPT * CHUNK
N_PAD = 10240
ROWS_PER_TILE = N_PAD // NS


def _mm_body(x_ref, w_ref, o_ref):
    o_ref[...] = jnp.dot(x_ref[...], w_ref[...], preferred_element_type=jnp.float32)


def _combine_body(p_ref, o_ref):
    o_ref[...] = jnp.maximum(p_ref[0] + p_ref[1], 0.0)


_sc_mesh = plsc.VectorSubcoreMesh(core_axis_name="c", subcore_axis_name="s")


@functools.partial(
    pl.kernel,
    mesh=_sc_mesh,
    out_type=jax.ShapeDtypeStruct((NC, N_PAD, D), jnp.float32),
    scratch_types=[
        pltpu.VMEM((HALF, CHUNK), jnp.int32),
        pltpu.VMEM((HALF, CHUNK), jnp.int32),
        pltpu.VMEM((HALF, CHUNK), jnp.float32),
        pltpu.VMEM((CHUNK, D), jnp.float32),
        pltpu.VMEM((CHUNK, D), jnp.float32),
        pltpu.VMEM_SHARED((N_PAD, D), jnp.float32),
        pltpu.SemaphoreType.DMA,
        pltpu.SemaphoreType.DMA,
        pltpu.SemaphoreType.DMA,
        pltpu.SemaphoreType.DMA,
    ],
)
def _sc_scatter(hidden_hbm, rows_hbm, cols_hbm, w_hbm, out_hbm,
                rows_v, cols_v, w_v, buf0, buf1, acc, gsem0, gsem1, ssem0, ssem1):
    c = lax.axis_index("c")
    s = lax.axis_index("s")
    wid = s * NC + c

    def zrow(i, carry):
        for j in range(D // 16):
            buf0[i, pl.ds(j * 16, 16)] = jnp.zeros((16,), jnp.float32)
        return carry

    lax.fori_loop(0, CHUNK, zrow, 0)
    for j in range(ROWS_PER_TILE // CHUNK):
        pltpu.sync_copy(
            buf0, acc.at[pl.ds(s * ROWS_PER_TILE + j * CHUNK, CHUNK)])
    plsc.subcore_barrier()

    bufs = (buf0, buf1)
    gsems = (gsem0, gsem1)
    ssems = (ssem0, ssem1)
    npair = HALF // 2

    for h in range(2):
        pltpu.sync_copy(rows_hbm.at[wid, pl.ds(h * HALF, HALF)], rows_v)
        pltpu.sync_copy(cols_hbm.at[wid, pl.ds(h * HALF, HALF)], cols_v)
        pltpu.sync_copy(w_hbm.at[wid, pl.ds(h * HALF, HALF)], w_v)

        for b in range(2):
            pltpu.async_copy(hidden_hbm.at[c].at[cols_v.at[b]], bufs[b], gsems[b])

        def pair_body(i, carry):
            for b in range(2):
                g = 2 * i + b
                buf = bufs[b]
                pltpu.make_async_copy(
                    hidden_hbm.at[c].at[cols_v.at[g]], buf, gsems[b]).wait()

                def group_body(k, carry2):
                    w16 = w_v[g, pl.ds(k * 16, 16)]
                    for el in range(16):
                        wsplat = lax.gather(
                            w16,
                            jnp.full((16, 1), el, jnp.int32),
                            lax.GatherDimensionNumbers(
                                offset_dims=(), collapsed_slice_dims=(0,),
                                start_index_map=(0,)),
                            slice_sizes=(1,),
                            mode=lax.GatherScatterMode.PROMISE_IN_BOUNDS,
                        )
                        e = k * 16 + el
                        for j in range(D // 16):
                            buf[e, pl.ds(j * 16, 16)] = (
                                wsplat * buf[e, pl.ds(j * 16, 16)])
                    return carry2

                lax.fori_loop(0, CHUNK // 16, group_body, 0)

                pltpu.async_copy(buf, acc.at[rows_v.at[g]], ssems[b], add=True)

                @pl.when(i < npair - 1)
                def _():
                    pltpu.make_async_copy(
                        buf, acc.at[rows_v.at[g]], ssems[b]).wait()
                    pltpu.async_copy(
                        hidden_hbm.at[c].at[cols_v.at[g + 2]], buf, gsems[b])
            return carry

        lax.fori_loop(0, npair, pair_body, 0)

        for b in range(2):
            g = HALF - 2 + b
            pltpu.make_async_copy(bufs[b], acc.at[rows_v.at[g]], ssems[b]).wait()

    plsc.subcore_barrier()

    plsc.subcore_barrier()

    pltpu.sync_copy(acc.at[pl.ds(s * ROWS_PER_TILE, ROWS_PER_TILE)],
                    out_hbm.at[c, pl.ds(s * ROWS_PER_TILE, ROWS_PER_TILE)])


def kernel(X, edge_index, edge_weight, W):
    X_flat = X.reshape(N_NODES, D)

    hidden = pl.pallas_call(
        _mm_body,
        grid=(10,),
        in_specs=[
            pl.BlockSpec((1000, D), lambda i: (i, 0)),
            pl.BlockSpec((D, D), lambda i: (0, 0)),
        ],
        out_specs=pl.BlockSpec((1000, D), lambda i: (i, 0)),
        out_shape=jax.ShapeDtypeStruct((N_NODES, D), jnp.float32),
    )(X_flat, W)

    e = edge_weight.shape[0]
    pad = E_PAD - e
    rows = jnp.concatenate(
        [edge_index[0].astype(jnp.int32), jnp.zeros((pad,), jnp.int32)]
    ).reshape(NW, CPT, CHUNK)
    cols = jnp.concatenate(
        [edge_index[1].astype(jnp.int32), jnp.zeros((pad,), jnp.int32)]
    ).reshape(NW, CPT, CHUNK)
    w_e = jnp.concatenate(
        [edge_weight.astype(jnp.float32), jnp.zeros((pad,), jnp.float32)]
    ).reshape(NW, CPT, CHUNK)
    partials = _sc_scatter(jnp.stack([hidden, hidden]), rows, cols, w_e)

    out = pl.pallas_call(
        _combine_body,
        grid=(10,),
        in_specs=[pl.BlockSpec((NC, 1024, D), lambda i: (0, i, 0))],
        out_specs=pl.BlockSpec((1024, D), lambda i: (i, 0)),
        out_shape=jax.ShapeDtypeStruct((N_PAD, D), jnp.float32),
    )(partials)

    return out[:N_NODES].reshape(1, N_NODES, D)


# speedup vs baseline: 1.3068x; 1.0157x over previous
import functools

import jax
import jax.numpy as jnp
from jax import lax
from jax.experimental import pallas as pl
from jax.experimental.pallas import tpu as pltpu
from jax.experimental.pallas import tpu_sc as plsc

N_NODES = 10000
D = 128
NC = 2
NS = 16
NW = NC * NS
CHUNK = 128
CPT = 80
HALF = CPT // 2
E_PAD = NW * CPT * CHUNK
N_PAD = 10240
ROWS_PER_TILE = N_PAD // NS


def _mm_body(x_ref, w_ref, o_ref):
    h = jnp.dot(x_ref[...], w_ref[...], preferred_element_type=jnp.float32)
    o_ref[0] = h
    o_ref[1] = h


def _combine_body(p_ref, o_ref):
    o_ref[...] = jnp.maximum(p_ref[0] + p_ref[1], 0.0)


_sc_mesh = plsc.VectorSubcoreMesh(core_axis_name="c", subcore_axis_name="s")


@functools.partial(
    pl.kernel,
    mesh=_sc_mesh,
    out_type=jax.ShapeDtypeStruct((NC, N_PAD, D), jnp.float32),
    scratch_types=[
        pltpu.VMEM((HALF, CHUNK), jnp.int32),
        pltpu.VMEM((HALF, CHUNK), jnp.int32),
        pltpu.VMEM((HALF, CHUNK), jnp.float32),
        pltpu.VMEM((CHUNK, D), jnp.float32),
        pltpu.VMEM((CHUNK, D), jnp.float32),
        pltpu.VMEM_SHARED((N_PAD, D), jnp.float32),
        pltpu.SemaphoreType.DMA,
        pltpu.SemaphoreType.DMA,
        pltpu.SemaphoreType.DMA,
        pltpu.SemaphoreType.DMA,
    ],
)
def _sc_scatter(hidden_hbm, rows_hbm, cols_hbm, w_hbm, out_hbm,
                rows_v, cols_v, w_v, buf0, buf1, acc, gsem0, gsem1, ssem0, ssem1):
    c = lax.axis_index("c")
    s = lax.axis_index("s")
    wid = s * NC + c

    def zrow(i, carry):
        for j in range(D // 16):
            buf0[i, pl.ds(j * 16, 16)] = jnp.zeros((16,), jnp.float32)
        return carry

    lax.fori_loop(0, CHUNK, zrow, 0)
    for j in range(ROWS_PER_TILE // CHUNK):
        pltpu.sync_copy(
            buf0, acc.at[pl.ds(s * ROWS_PER_TILE + j * CHUNK, CHUNK)])
    plsc.subcore_barrier()

    bufs = (buf0, buf1)
    gsems = (gsem0, gsem1)
    ssems = (ssem0, ssem1)
    npair = HALF // 2

    for h in range(2):
        pltpu.sync_copy(rows_hbm.at[wid, pl.ds(h * HALF, HALF)], rows_v)
        pltpu.sync_copy(cols_hbm.at[wid, pl.ds(h * HALF, HALF)], cols_v)
        pltpu.sync_copy(w_hbm.at[wid, pl.ds(h * HALF, HALF)], w_v)

        for b in range(2):
            pltpu.async_copy(hidden_hbm.at[c].at[cols_v.at[b]], bufs[b], gsems[b])

        def pair_body(i, carry):
            for b in range(2):
                g = 2 * i + b
                buf = bufs[b]
                pltpu.make_async_copy(
                    hidden_hbm.at[c].at[cols_v.at[g]], buf, gsems[b]).wait()

                def group_body(k, carry2):
                    w16 = w_v[g, pl.ds(k * 16, 16)]
                    for el in range(16):
                        wsplat = lax.gather(
                            w16,
                            jnp.full((16, 1), el, jnp.int32),
                            lax.GatherDimensionNumbers(
                                offset_dims=(), collapsed_slice_dims=(0,),
                                start_index_map=(0,)),
                            slice_sizes=(1,),
                            mode=lax.GatherScatterMode.PROMISE_IN_BOUNDS,
                        )
                        e = k * 16 + el
                        for j in range(D // 16):
                            buf[e, pl.ds(j * 16, 16)] = (
                                wsplat * buf[e, pl.ds(j * 16, 16)])
                    return carry2

                lax.fori_loop(0, CHUNK // 16, group_body, 0)

                pltpu.async_copy(buf, acc.at[rows_v.at[g]], ssems[b], add=True)

                @pl.when(i < npair - 1)
                def _():
                    pltpu.make_async_copy(
                        buf, acc.at[rows_v.at[g]], ssems[b]).wait()
                    pltpu.async_copy(
                        hidden_hbm.at[c].at[cols_v.at[g + 2]], buf, gsems[b])
            return carry

        lax.fori_loop(0, npair, pair_body, 0)

        for b in range(2):
            g = HALF - 2 + b
            pltpu.make_async_copy(bufs[b], acc.at[rows_v.at[g]], ssems[b]).wait()

    plsc.subcore_barrier()

    plsc.subcore_barrier()

    pltpu.sync_copy(acc.at[pl.ds(s * ROWS_PER_TILE, ROWS_PER_TILE)],
                    out_hbm.at[c, pl.ds(s * ROWS_PER_TILE, ROWS_PER_TILE)])


def kernel(X, edge_index, edge_weight, W):
    X_flat = X.reshape(N_NODES, D)

    hidden2 = pl.pallas_call(
        _mm_body,
        grid=(10,),
        in_specs=[
            pl.BlockSpec((1000, D), lambda i: (i, 0)),
            pl.BlockSpec((D, D), lambda i: (0, 0)),
        ],
        out_specs=pl.BlockSpec((NC, 1000, D), lambda i: (0, i, 0)),
        out_shape=jax.ShapeDtypeStruct((NC, N_NODES, D), jnp.float32),
    )(X_flat, W)

    e = edge_weight.shape[0]
    pad = E_PAD - e
    rows = jnp.concatenate(
        [edge_index[0].astype(jnp.int32), jnp.zeros((pad,), jnp.int32)]
    ).reshape(NW, CPT, CHUNK)
    cols = jnp.concatenate(
        [edge_index[1].astype(jnp.int32), jnp.zeros((pad,), jnp.int32)]
    ).reshape(NW, CPT, CHUNK)
    w_e = jnp.concatenate(
        [edge_weight.astype(jnp.float32), jnp.zeros((pad,), jnp.float32)]
    ).reshape(NW, CPT, CHUNK)
    partials = _sc_scatter(hidden2, rows, cols, w_e)

    out = pl.pallas_call(
        _combine_body,
        grid=(10,),
        in_specs=[pl.BlockSpec((NC, 1024, D), lambda i: (0, i, 0))],
        out_specs=pl.BlockSpec((1024, D), lambda i: (i, 0)),
        out_shape=jax.ShapeDtypeStruct((N_PAD, D), jnp.float32),
    )(partials)

    return out[:N_NODES].reshape(1, N_NODES, D)
